# baseline (speedup 1.0000x reference)
import jax
import jax.numpy as jnp
from jax import lax
from jax.experimental import pallas as pl
from jax.experimental.pallas import tpu as pltpu
from jax.experimental.pallas import tpu_sc as plsc

_ALT_VALUES = (150, 200, 250, 300)
_NUM_PARTS = 16
_NUM_ROWS = 5
_BATCH = 16384
_NC, _NS = 2, 16
_NW = _NC * _NS
_BPW = _BATCH // _NW
_GROUPS = _BPW // 16


def _sc_kernel(att_hbm, alt_hbm, temp_hbm, out_hbm, att_v, temp_v, alt_v,
               table_v, rows_v):
    wid = lax.axis_index("s") * _NC + lax.axis_index("c")
    base = wid * _BPW
    pltpu.sync_copy(att_hbm, att_v)
    pltpu.sync_copy(temp_hbm, temp_v)
    pltpu.sync_copy(alt_hbm.at[pl.ds(base, _BPW)], alt_v)

    recip = 1.0 / jnp.maximum(jnp.abs(temp_v[...]), jnp.float32(0.1))
    for i in range(_NUM_ROWS):
        w = att_v[i] * recip
        e = jnp.exp(w - jnp.max(w))
        table_v[pl.ds(i * _NUM_PARTS, _NUM_PARTS)] = e / jnp.sum(e)

    lane = lax.iota(jnp.int32, 16)
    for g in range(_GROUPS):
        a = alt_v[pl.ds(g * 16, 16)]
        idx = jnp.full((16,), 4, dtype=jnp.int32)
        for i, v in enumerate(_ALT_VALUES):
            idx = jnp.where(a == jnp.int32(v), jnp.int32(i), idx)
        src = idx * _NUM_PARTS
        rid = g * 16 + lane
        for l in range(_NUM_PARTS):
            col = plsc.load_gather(table_v, [src + l])
            plsc.store_scatter(rows_v, [rid, jnp.full((16,), l, jnp.int32)],
                               col)

    pltpu.sync_copy(rows_v, out_hbm.at[pl.ds(base, _BPW)])


def kernel(altitudes, attention, temp):
    mesh = plsc.VectorSubcoreMesh(core_axis_name="c", subcore_axis_name="s")
    run = pl.kernel(
        _sc_kernel,
        out_type=jax.ShapeDtypeStruct((_BATCH, _NUM_PARTS), jnp.float32),
        mesh=mesh,
        compiler_params=pltpu.CompilerParams(
            use_tc_tiling_on_sc=True, needs_layout_passes=False,
            skip_device_barrier=True),
        scratch_types=[
            pltpu.VMEM((_NUM_ROWS, _NUM_PARTS), jnp.float32),
            pltpu.VMEM((16,), jnp.float32),
            pltpu.VMEM((_BPW,), jnp.int32),
            pltpu.VMEM((_NUM_ROWS * _NUM_PARTS,), jnp.float32),
            pltpu.VMEM((_BPW, _NUM_PARTS), jnp.float32),
        ],
    )
    temp16 = jnp.broadcast_to(jnp.asarray(temp, jnp.float32).reshape(1), (16,))
    return run(attention, altitudes, temp16)


# speedup vs baseline: 1.0323x; 1.0323x over previous
import jax
import jax.numpy as jnp
from jax import lax
from jax.experimental import pallas as pl
from jax.experimental.pallas import tpu as pltpu
from jax.experimental.pallas import tpu_sc as plsc

_ALT_VALUES = (150, 200, 250, 300)
_NUM_PARTS = 16
_NUM_ROWS = 5
_BATCH = 16384
_NC, _NS = 2, 16
_NW = _NC * _NS
_BPW = _BATCH // _NW
_GROUPS = _BPW // 16


_NCHUNK = 4
_ROWS_PER_CHUNK = _BPW // _NCHUNK
_GROUPS_PER_CHUNK = _ROWS_PER_CHUNK // 16


def _sc_kernel(att_hbm, alt_hbm, temp_hbm, out_hbm, att_v, temp_v, alt_v,
               table_v, rows_v, sems):
    wid = lax.axis_index("s") * _NC + lax.axis_index("c")
    base = wid * _BPW
    pltpu.sync_copy(att_hbm, att_v)
    pltpu.sync_copy(temp_hbm, temp_v)
    pltpu.sync_copy(alt_hbm.at[pl.ds(base, _BPW)], alt_v)

    recip = 1.0 / jnp.maximum(jnp.abs(temp_v[...]), jnp.float32(0.1))
    for i in range(_NUM_ROWS):
        w = att_v[i] * recip
        e = jnp.exp(w - jnp.max(w))
        table_v[pl.ds(i * _NUM_PARTS, _NUM_PARTS)] = e / jnp.sum(e)

    lane = lax.iota(jnp.int32, 16)
    lfull = [jnp.full((16,), l, jnp.int32) for l in range(_NUM_PARTS)]
    copies = [None, None]
    for c in range(_NCHUNK):
        b = c % 2
        if copies[b] is not None:
            copies[b].wait()
        for gg in range(_GROUPS_PER_CHUNK):
            g = c * _GROUPS_PER_CHUNK + gg
            a = alt_v[pl.ds(g * 16, 16)]
            idx = jnp.full((16,), 4, dtype=jnp.int32)
            for i, v in enumerate(_ALT_VALUES):
                idx = jnp.where(a == jnp.int32(v), jnp.int32(i), idx)
            src = idx * _NUM_PARTS
            rid = gg * 16 + lane
            for l in range(_NUM_PARTS):
                col = plsc.load_gather(table_v, [src + l])
                plsc.store_scatter(rows_v.at[b], [rid, lfull[l]], col)
        copies[b] = pltpu.async_copy(
            rows_v.at[b],
            out_hbm.at[pl.ds(base + c * _ROWS_PER_CHUNK, _ROWS_PER_CHUNK)],
            sems.at[b],
        )
    for b in range(2):
        copies[b].wait()


def kernel(altitudes, attention, temp):
    mesh = plsc.VectorSubcoreMesh(core_axis_name="c", subcore_axis_name="s")
    run = pl.kernel(
        _sc_kernel,
        out_type=jax.ShapeDtypeStruct((_BATCH, _NUM_PARTS), jnp.float32),
        mesh=mesh,
        compiler_params=pltpu.CompilerParams(
            use_tc_tiling_on_sc=True, needs_layout_passes=False),
        scratch_types=[
            pltpu.VMEM((_NUM_ROWS, _NUM_PARTS), jnp.float32),
            pltpu.VMEM((16,), jnp.float32),
            pltpu.VMEM((_BPW,), jnp.int32),
            pltpu.VMEM((_NUM_ROWS * _NUM_PARTS,), jnp.float32),
            pltpu.VMEM((2, _ROWS_PER_CHUNK, _NUM_PARTS), jnp.float32),
            pltpu.SemaphoreType.DMA((2,)),
        ],
    )
    temp16 = jnp.broadcast_to(jnp.asarray(temp, jnp.float32).reshape(1), (16,))
    return run(attention, altitudes, temp16)


# speedup vs baseline: 1.1651x; 1.1286x over previous
import jax
import jax.numpy as jnp
from jax import lax
from jax.experimental import pallas as pl
from jax.experimental.pallas import tpu as pltpu
from jax.experimental.pallas import tpu_sc as plsc

_ALT_VALUES = (150, 200, 250, 300)
_NUM_PARTS = 16
_NUM_ROWS = 5
_BATCH = 16384
_NC, _NS = 2, 16
_NW = _NC * _NS
_BPW = _BATCH // _NW
_GROUPS = _BPW // 16


_NCHUNK = 4
_ROWS_PER_CHUNK = _BPW // _NCHUNK
_GROUPS_PER_CHUNK = _ROWS_PER_CHUNK // 16


def _sc_kernel(att_hbm, alt_hbm, temp_hbm, out_hbm, att_v, temp_v, alt_v,
               table_v, rows_v, sems):
    wid = lax.axis_index("s") * _NC + lax.axis_index("c")
    base = wid * _BPW
    pltpu.sync_copy(att_hbm, att_v)
    pltpu.sync_copy(temp_hbm, temp_v)
    pltpu.sync_copy(alt_hbm.at[pl.ds(base, _BPW)], alt_v)

    recip = 1.0 / jnp.maximum(jnp.abs(temp_v[...]), jnp.float32(0.1))
    for i in range(_NUM_ROWS):
        w = att_v[i] * recip
        e = jnp.exp(w - jnp.max(w))
        table_v[pl.ds(i * _NUM_PARTS, _NUM_PARTS)] = e / jnp.sum(e)

    copies = [None, None]
    for c in range(_NCHUNK):
        b = c % 2
        if copies[b] is not None:
            copies[b].wait()
        for gg in range(_GROUPS_PER_CHUNK):
            g = c * _GROUPS_PER_CHUNK + gg
            a = alt_v[pl.ds(g * 16, 16)]
            idx = jnp.full((16,), 4, dtype=jnp.int32)
            for i, v in enumerate(_ALT_VALUES):
                idx = jnp.where(a == jnp.int32(v), jnp.int32(i), idx)
            offs = idx * _NUM_PARTS
            for k in range(16):
                rows_v[b, gg * 16 + k, :] = table_v[pl.ds(offs[k],
                                                          _NUM_PARTS)]
        copies[b] = pltpu.async_copy(
            rows_v.at[b],
            out_hbm.at[pl.ds(base + c * _ROWS_PER_CHUNK, _ROWS_PER_CHUNK)],
            sems.at[b],
        )
    for b in range(2):
        copies[b].wait()


def kernel(altitudes, attention, temp):
    mesh = plsc.VectorSubcoreMesh(core_axis_name="c", subcore_axis_name="s")
    run = pl.kernel(
        _sc_kernel,
        out_type=jax.ShapeDtypeStruct((_BATCH, _NUM_PARTS), jnp.float32),
        mesh=mesh,
        compiler_params=pltpu.CompilerParams(
            use_tc_tiling_on_sc=True, needs_layout_passes=False),
        scratch_types=[
            pltpu.VMEM((_NUM_ROWS, _NUM_PARTS), jnp.float32),
            pltpu.VMEM((16,), jnp.float32),
            pltpu.VMEM((_BPW,), jnp.int32),
            pltpu.VMEM((_NUM_ROWS * _NUM_PARTS,), jnp.float32),
            pltpu.VMEM((2, _ROWS_PER_CHUNK, _NUM_PARTS), jnp.float32),
            pltpu.SemaphoreType.DMA((2,)),
        ],
    )
    temp16 = jnp.broadcast_to(jnp.asarray(temp, jnp.float32).reshape(1), (16,))
    return run(attention, altitudes, temp16)


# speedup vs baseline: 1.2245x; 1.0510x over previous
import jax
import jax.numpy as jnp
from jax import lax
from jax.experimental import pallas as pl
from jax.experimental.pallas import tpu as pltpu
from jax.experimental.pallas import tpu_sc as plsc

_ALT_VALUES = (150, 200, 250, 300)
_NUM_PARTS = 16
_NUM_ROWS = 5
_BATCH = 16384
_NC, _NS = 2, 16
_NW = _NC * _NS
_BPW = _BATCH // _NW
_GROUPS = _BPW // 16


_NCHUNK = 4
_ROWS_PER_CHUNK = _BPW // _NCHUNK
_GROUPS_PER_CHUNK = _ROWS_PER_CHUNK // 16


def _sc_kernel(att_hbm, alt_hbm, temp_hbm, out_hbm, att_v, temp_v, alt_v,
               table_v, rows_v, sems):
    wid = lax.axis_index("s") * _NC + lax.axis_index("c")
    base = wid * _BPW
    pltpu.sync_copy(att_hbm, att_v)
    pltpu.sync_copy(temp_hbm, temp_v)
    pltpu.sync_copy(alt_hbm.at[pl.ds(base, _BPW)], alt_v)

    recip = 1.0 / jnp.maximum(jnp.abs(temp_v[...]), jnp.float32(0.1))
    for i in range(_NUM_ROWS):
        w = att_v[i] * recip
        e = jnp.exp(w - jnp.max(w))
        table_v[pl.ds(i * _NUM_PARTS, _NUM_PARTS)] = e / jnp.sum(e)

    copies = [None, None]
    for c in range(_NCHUNK):
        b = c % 2
        if copies[b] is not None:
            copies[b].wait()

        def group_body(gg, carry, c=c, b=b):
            a = alt_v[pl.ds((c * _GROUPS_PER_CHUNK + gg) * 16, 16)]
            idx = jnp.full((16,), 4, dtype=jnp.int32)
            for i, v in enumerate(_ALT_VALUES):
                idx = jnp.where(a == jnp.int32(v), jnp.int32(i), idx)
            offs = idx * _NUM_PARTS
            for k in range(16):
                rows_v[b, gg * 16 + k, :] = table_v[pl.ds(offs[k],
                                                          _NUM_PARTS)]
            return carry

        lax.fori_loop(0, _GROUPS_PER_CHUNK, group_body, 0, unroll=2)
        copies[b] = pltpu.async_copy(
            rows_v.at[b],
            out_hbm.at[pl.ds(base + c * _ROWS_PER_CHUNK, _ROWS_PER_CHUNK)],
            sems.at[b],
        )
    for b in range(2):
        copies[b].wait()


def kernel(altitudes, attention, temp):
    mesh = plsc.VectorSubcoreMesh(core_axis_name="c", subcore_axis_name="s")
    run = pl.kernel(
        _sc_kernel,
        out_type=jax.ShapeDtypeStruct((_BATCH, _NUM_PARTS), jnp.float32),
        mesh=mesh,
        compiler_params=pltpu.CompilerParams(
            use_tc_tiling_on_sc=True, needs_layout_passes=False),
        scratch_types=[
            pltpu.VMEM((_NUM_ROWS, _NUM_PARTS), jnp.float32),
            pltpu.VMEM((16,), jnp.float32),
            pltpu.VMEM((_BPW,), jnp.int32),
            pltpu.VMEM((_NUM_ROWS * _NUM_PARTS,), jnp.float32),
            pltpu.VMEM((2, _ROWS_PER_CHUNK, _NUM_PARTS), jnp.float32),
            pltpu.SemaphoreType.DMA((2,)),
        ],
    )
    temp16 = jnp.broadcast_to(jnp.asarray(temp, jnp.float32).reshape(1), (16,))
    return run(attention, altitudes, temp16)


# speedup vs baseline: 1.2633x; 1.0317x over previous
import jax
import jax.numpy as jnp
from jax import lax
from jax.experimental import pallas as pl
from jax.experimental.pallas import tpu as pltpu
from jax.experimental.pallas import tpu_sc as plsc

_ALT_VALUES = (150, 200, 250, 300)
_NUM_PARTS = 16
_NUM_ROWS = 5
_BATCH = 16384
_NC, _NS = 2, 16
_NW = _NC * _NS
_BPW = _BATCH // _NW
_GROUPS = _BPW // 16


_NCHUNK = 4
_ROWS_PER_CHUNK = _BPW // _NCHUNK
_GROUPS_PER_CHUNK = _ROWS_PER_CHUNK // 16


def _sc_kernel(att_hbm, alt_hbm, temp_hbm, out_hbm, att_v, temp_v, alt_v,
               table_v, rows_v, sems):
    wid = lax.axis_index("s") * _NC + lax.axis_index("c")
    base = wid * _BPW
    pltpu.sync_copy(att_hbm, att_v)
    pltpu.sync_copy(temp_hbm, temp_v)
    pltpu.sync_copy(alt_hbm.at[pl.ds(base, _BPW)], alt_v)

    recip = 1.0 / jnp.maximum(jnp.abs(temp_v[...]), jnp.float32(0.1))
    for i in range(_NUM_ROWS):
        w = att_v[i] * recip
        e = jnp.exp(w - jnp.max(w))
        table_v[pl.ds(i * _NUM_PARTS, _NUM_PARTS)] = e / jnp.sum(e)

    default_row = table_v[pl.ds(4 * _NUM_PARTS, _NUM_PARTS)]
    copies = [None, None]
    for c in range(_NCHUNK):
        b = c % 2
        if copies[b] is not None:
            copies[b].wait()

        def fill_body(e, carry, b=b):
            rows_v[b, e, :] = default_row
            return carry

        lax.fori_loop(0, _ROWS_PER_CHUNK, fill_body, 0, unroll=4)

        def group_body(gg, carry, c=c, b=b):
            a = alt_v[pl.ds((c * _GROUPS_PER_CHUNK + gg) * 16, 16)]
            hits = (a == jnp.int32(_ALT_VALUES[0]))
            for v in _ALT_VALUES[1:]:
                hits = hits | (a == jnp.int32(v))

            @pl.when(jnp.any(hits))
            def _fixup():
                idx = jnp.full((16,), 4, dtype=jnp.int32)
                for i, v in enumerate(_ALT_VALUES):
                    idx = jnp.where(a == jnp.int32(v), jnp.int32(i), idx)
                offs = idx * _NUM_PARTS
                for k in range(16):
                    rows_v[b, gg * 16 + k, :] = table_v[pl.ds(offs[k],
                                                              _NUM_PARTS)]
            return carry

        lax.fori_loop(0, _GROUPS_PER_CHUNK, group_body, 0)
        copies[b] = pltpu.async_copy(
            rows_v.at[b],
            out_hbm.at[pl.ds(base + c * _ROWS_PER_CHUNK, _ROWS_PER_CHUNK)],
            sems.at[b],
        )
    for b in range(2):
        copies[b].wait()


def kernel(altitudes, attention, temp):
    mesh = plsc.VectorSubcoreMesh(core_axis_name="c", subcore_axis_name="s")
    run = pl.kernel(
        _sc_kernel,
        out_type=jax.ShapeDtypeStruct((_BATCH, _NUM_PARTS), jnp.float32),
        mesh=mesh,
        compiler_params=pltpu.CompilerParams(
            use_tc_tiling_on_sc=True, needs_layout_passes=False),
        scratch_types=[
            pltpu.VMEM((_NUM_ROWS, _NUM_PARTS), jnp.float32),
            pltpu.VMEM((16,), jnp.float32),
            pltpu.VMEM((_BPW,), jnp.int32),
            pltpu.VMEM((_NUM_ROWS * _NUM_PARTS,), jnp.float32),
            pltpu.VMEM((2, _ROWS_PER_CHUNK, _NUM_PARTS), jnp.float32),
            pltpu.SemaphoreType.DMA((2,)),
        ],
    )
    temp16 = jnp.broadcast_to(jnp.asarray(temp, jnp.float32).reshape(1), (16,))
    return run(attention, altitudes, temp16)


# speedup vs baseline: 1.2882x; 1.0197x over previous
import jax
import jax.numpy as jnp
from jax import lax
from jax.experimental import pallas as pl
from jax.experimental.pallas import tpu as pltpu
from jax.experimental.pallas import tpu_sc as plsc

_ALT_VALUES = (150, 200, 250, 300)
_NUM_PARTS = 16
_NUM_ROWS = 5
_BATCH = 16384
_NC, _NS = 2, 16
_NW = _NC * _NS
_BPW = _BATCH // _NW
_GROUPS = _BPW // 16


_NCHUNK = 4
_ROWS_PER_CHUNK = _BPW // _NCHUNK
_GROUPS_PER_CHUNK = _ROWS_PER_CHUNK // 16


def _sc_kernel(att_hbm, alt_hbm, temp_hbm, out_hbm, att_v, temp_v, alt_v,
               table_v, rows_v, sems):
    wid = lax.axis_index("s") * _NC + lax.axis_index("c")
    base = wid * _BPW
    pltpu.sync_copy(att_hbm, att_v)
    pltpu.sync_copy(temp_hbm, temp_v)
    pltpu.sync_copy(alt_hbm.at[pl.ds(base, _BPW)], alt_v)

    recip = 1.0 / jnp.maximum(jnp.abs(temp_v[...]), jnp.float32(0.1))
    for i in range(_NUM_ROWS):
        w = att_v[i] * recip
        e = jnp.exp(w - jnp.max(w))
        table_v[pl.ds(i * _NUM_PARTS, _NUM_PARTS)] = e / jnp.sum(e)

    default_row = table_v[pl.ds(4 * _NUM_PARTS, _NUM_PARTS)]

    def chunk_body(c, carry):
        def fill_body(e, carry2):
            rows_v[c * _ROWS_PER_CHUNK + e, :] = default_row
            return carry2

        lax.fori_loop(0, _ROWS_PER_CHUNK, fill_body, 0, unroll=4)

        def group_body(gg, carry2):
            g = c * _GROUPS_PER_CHUNK + gg
            a = alt_v[pl.ds(g * 16, 16)]
            hits = (a == jnp.int32(_ALT_VALUES[0]))
            for v in _ALT_VALUES[1:]:
                hits = hits | (a == jnp.int32(v))

            @pl.when(jnp.any(hits))
            def _fixup():
                idx = jnp.full((16,), 4, dtype=jnp.int32)
                for i, v in enumerate(_ALT_VALUES):
                    idx = jnp.where(a == jnp.int32(v), jnp.int32(i), idx)
                offs = idx * _NUM_PARTS
                for k in range(16):
                    rows_v[g * 16 + k, :] = table_v[pl.ds(offs[k],
                                                          _NUM_PARTS)]
            return carry2

        lax.fori_loop(0, _GROUPS_PER_CHUNK, group_body, 0)
        pltpu.async_copy(
            rows_v.at[pl.ds(c * _ROWS_PER_CHUNK, _ROWS_PER_CHUNK)],
            out_hbm.at[pl.ds(base + c * _ROWS_PER_CHUNK, _ROWS_PER_CHUNK)],
            sems,
        )
        return carry

    lax.fori_loop(0, _NCHUNK, chunk_body, 0)
    for _ in range(_NCHUNK):
        pltpu.make_async_copy(
            rows_v.at[pl.ds(0, _ROWS_PER_CHUNK)],
            out_hbm.at[pl.ds(base, _ROWS_PER_CHUNK)],
            sems,
        ).wait()


def kernel(altitudes, attention, temp):
    mesh = plsc.VectorSubcoreMesh(core_axis_name="c", subcore_axis_name="s")
    run = pl.kernel(
        _sc_kernel,
        out_type=jax.ShapeDtypeStruct((_BATCH, _NUM_PARTS), jnp.float32),
        mesh=mesh,
        compiler_params=pltpu.CompilerParams(
            use_tc_tiling_on_sc=True, needs_layout_passes=False),
        scratch_types=[
            pltpu.VMEM((_NUM_ROWS, _NUM_PARTS), jnp.float32),
            pltpu.VMEM((16,), jnp.float32),
            pltpu.VMEM((_BPW,), jnp.int32),
            pltpu.VMEM((_NUM_ROWS * _NUM_PARTS,), jnp.float32),
            pltpu.VMEM((_BPW, _NUM_PARTS), jnp.float32),
            pltpu.SemaphoreType.DMA,
        ],
    )
    temp16 = jnp.broadcast_to(jnp.asarray(temp, jnp.float32).reshape(1), (16,))
    return run(attention, altitudes, temp16)


# speedup vs baseline: 1.2915x; 1.0025x over previous
import jax
import jax.numpy as jnp
from jax import lax
from jax.experimental import pallas as pl
from jax.experimental.pallas import tpu as pltpu
from jax.experimental.pallas import tpu_sc as plsc

_ALT_VALUES = (150, 200, 250, 300)
_NUM_PARTS = 16
_NUM_ROWS = 5
_BATCH = 16384
_NC, _NS = 2, 16
_NW = _NC * _NS
_BPW = _BATCH // _NW
_GROUPS = _BPW // 16


_NCHUNK = 8
_ROWS_PER_CHUNK = _BPW // _NCHUNK
_GROUPS_PER_CHUNK = _ROWS_PER_CHUNK // 16


def _sc_kernel(att_hbm, alt_hbm, temp_hbm, out_hbm, att_v, temp_v, alt_v,
               table_v, rows_v, sems):
    wid = lax.axis_index("s") * _NC + lax.axis_index("c")
    base = wid * _BPW
    pltpu.sync_copy(att_hbm, att_v)
    pltpu.sync_copy(temp_hbm, temp_v)
    pltpu.sync_copy(alt_hbm.at[pl.ds(base, _BPW)], alt_v)

    recip = 1.0 / jnp.maximum(jnp.abs(temp_v[...]), jnp.float32(0.1))
    for i in range(_NUM_ROWS):
        w = att_v[i] * recip
        e = jnp.exp(w - jnp.max(w))
        table_v[pl.ds(i * _NUM_PARTS, _NUM_PARTS)] = e / jnp.sum(e)

    default_row = table_v[pl.ds(4 * _NUM_PARTS, _NUM_PARTS)]

    def chunk_body(c, carry):
        def fill_body(e, carry2):
            rows_v[c * _ROWS_PER_CHUNK + e, :] = default_row
            return carry2

        lax.fori_loop(0, _ROWS_PER_CHUNK, fill_body, 0, unroll=4)

        def group_body(gg, carry2):
            g = c * _GROUPS_PER_CHUNK + gg
            a = alt_v[pl.ds(g * 16, 16)]
            hits = (a == jnp.int32(_ALT_VALUES[0]))
            for v in _ALT_VALUES[1:]:
                hits = hits | (a == jnp.int32(v))

            @pl.when(jnp.any(hits))
            def _fixup():
                idx = jnp.full((16,), 4, dtype=jnp.int32)
                for i, v in enumerate(_ALT_VALUES):
                    idx = jnp.where(a == jnp.int32(v), jnp.int32(i), idx)
                offs = idx * _NUM_PARTS
                for k in range(16):
                    rows_v[g * 16 + k, :] = table_v[pl.ds(offs[k],
                                                          _NUM_PARTS)]
            return carry2

        lax.fori_loop(0, _GROUPS_PER_CHUNK, group_body, 0)
        pltpu.async_copy(
            rows_v.at[pl.ds(c * _ROWS_PER_CHUNK, _ROWS_PER_CHUNK)],
            out_hbm.at[pl.ds(base + c * _ROWS_PER_CHUNK, _ROWS_PER_CHUNK)],
            sems,
        )
        return carry

    lax.fori_loop(0, _NCHUNK, chunk_body, 0)
    for _ in range(_NCHUNK):
        pltpu.make_async_copy(
            rows_v.at[pl.ds(0, _ROWS_PER_CHUNK)],
            out_hbm.at[pl.ds(base, _ROWS_PER_CHUNK)],
            sems,
        ).wait()


def kernel(altitudes, attention, temp):
    mesh = plsc.VectorSubcoreMesh(core_axis_name="c", subcore_axis_name="s")
    run = pl.kernel(
        _sc_kernel,
        out_type=jax.ShapeDtypeStruct((_BATCH, _NUM_PARTS), jnp.float32),
        mesh=mesh,
        compiler_params=pltpu.CompilerParams(
            use_tc_tiling_on_sc=True, needs_layout_passes=False),
        scratch_types=[
            pltpu.VMEM((_NUM_ROWS, _NUM_PARTS), jnp.float32),
            pltpu.VMEM((16,), jnp.float32),
            pltpu.VMEM((_BPW,), jnp.int32),
            pltpu.VMEM((_NUM_ROWS * _NUM_PARTS,), jnp.float32),
            pltpu.VMEM((_BPW, _NUM_PARTS), jnp.float32),
            pltpu.SemaphoreType.DMA,
        ],
    )
    temp16 = jnp.broadcast_to(jnp.asarray(temp, jnp.float32).reshape(1), (16,))
    return run(attention, altitudes, temp16)
